# baseline (speedup 1.0000x reference)
import functools

import jax
import jax.numpy as jnp
from jax.experimental import pallas as pl
from jax.experimental.pallas import tpu as pltpu

_INT_MIN = -2147483648
_INT_MAX = 2147483647


def _sproj_body(sents_ref, wb_ref, bias_ref, out_ref, *, s, sp):
    out_ref[0, :s] = jnp.dot(sents_ref[0], wb_ref[...],
                             preferred_element_type=jnp.float32)
    out_ref[0, s:s + 1] = bias_ref[...]
    out_ref[0, s + 1:] = jnp.zeros((sp - s - 1, out_ref.shape[-1]),
                                   jnp.float32)


def _main_body(meta_ref, words_ref, sproj_ref, wa_ref, out_ref, *, tl, sp):
    j = pl.program_id(1)
    lo = meta_ref[0, 0:1, :sp]
    hi = meta_ref[0, 1:2, :sp]
    t = j * tl + jax.lax.broadcasted_iota(jnp.int32, (tl, sp), 0)
    coef = ((t >= lo) & (t < hi)).astype(jnp.float32)
    acc = jnp.dot(words_ref[0].astype(jnp.bfloat16), wa_ref[...],
                  preferred_element_type=jnp.float32)
    acc += jnp.dot(coef, sproj_ref[0], preferred_element_type=jnp.float32)
    out_ref[0] = jnp.maximum(acc, 0.0)


def kernel(words_emb, sents_emb, batch_bound_sents, W1_weight, W1_bias):
    B, L, D = words_emb.shape
    S = sents_emb.shape[1]
    TL = 2048
    SP = 64
    META_LANES = 128

    wa = W1_weight[:, :D].T.astype(jnp.bfloat16)
    wb = W1_weight[:, D:].T

    starts = batch_bound_sents[:, :, 0]
    ebp1 = batch_bound_sents[:, -1, 1:2] + 1
    lo = jnp.full((B, META_LANES), _INT_MAX, jnp.int32)
    lo = lo.at[:, :S].set(starts).at[:, S].set(_INT_MIN)
    hi = jnp.full((B, META_LANES), _INT_MAX, jnp.int32)
    hi = hi.at[:, :S - 1].set(starts[:, 1:]).at[:, S - 1].set(ebp1[:, 0])
    meta = jnp.stack([lo, hi], axis=1)

    sprojp = pl.pallas_call(
        functools.partial(_sproj_body, s=S, sp=SP),
        grid=(B,),
        in_specs=[
            pl.BlockSpec((1, S, D), lambda b: (b, 0, 0)),
            pl.BlockSpec((D, D), lambda b: (0, 0)),
            pl.BlockSpec((1, D), lambda b: (0, 0)),
        ],
        out_specs=pl.BlockSpec((1, SP, D), lambda b: (b, 0, 0)),
        out_shape=jax.ShapeDtypeStruct((B, SP, D), jnp.float32),
    )(sents_emb, wb, W1_bias.reshape(1, D))

    out = pl.pallas_call(
        functools.partial(_main_body, tl=TL, sp=SP),
        grid=(B, L // TL),
        in_specs=[
            pl.BlockSpec((1, 2, META_LANES), lambda b, j: (b, 0, 0)),
            pl.BlockSpec((1, TL, D), lambda b, j: (b, j, 0)),
            pl.BlockSpec((1, SP, D), lambda b, j: (b, 0, 0)),
            pl.BlockSpec((D, D), lambda b, j: (0, 0)),
        ],
        out_specs=pl.BlockSpec((1, TL, D), lambda b, j: (b, j, 0)),
        out_shape=jax.ShapeDtypeStruct((B, L, D), jnp.float32),
    )(meta, words_emb, sprojp, wa)
    return out


# speedup vs baseline: 1.1129x; 1.1129x over previous
import functools

import jax
import jax.numpy as jnp
from jax.experimental import pallas as pl


def _sproj_body(sents_ref, wb_ref, out_ref):
    out_ref[...] = jnp.dot(sents_ref[...], wb_ref[...],
                           preferred_element_type=jnp.float32)


def _main_body(meta_ref, words_ref, sproj_ref, wa_ref, bias_ref, out_ref,
               *, tl, s):
    j = pl.program_id(1)
    row = meta_ref[0]
    starts = jax.lax.slice(row, (0, 0), (1, s))
    nxt = jax.lax.slice(row, (0, 1), (1, s + 1))
    t = j * tl + jax.lax.broadcasted_iota(jnp.int32, (tl, s), 0)
    coef = ((t >= starts) & (t < nxt)).astype(jnp.float32)
    acc = jnp.dot(words_ref[0].astype(jnp.bfloat16), wa_ref[...],
                  preferred_element_type=jnp.float32)
    acc += jnp.dot(coef, sproj_ref[0],
                   preferred_element_type=jnp.float32)
    out_ref[0] = jnp.maximum(acc + bias_ref[...], 0.0)


def kernel(words_emb, sents_emb, batch_bound_sents, W1_weight, W1_bias):
    B, L, D = words_emb.shape
    S = sents_emb.shape[1]
    TL = 2048
    META_LANES = 128

    wa = W1_weight[:, :D].T.astype(jnp.bfloat16)
    wb = W1_weight[:, D:].T
    bias = W1_bias.reshape(1, D)

    starts = batch_bound_sents[:, :, 0]
    ebp1 = batch_bound_sents[:, -1, 1:2] + 1
    meta = jnp.zeros((B, 1, META_LANES), jnp.int32)
    meta = meta.at[:, 0, :S].set(starts).at[:, 0, S].set(ebp1[:, 0])

    sproj = pl.pallas_call(
        _sproj_body,
        out_shape=jax.ShapeDtypeStruct((B * S, D), jnp.float32),
    )(sents_emb.reshape(B * S, D), wb).reshape(B, S, D)

    out = pl.pallas_call(
        functools.partial(_main_body, tl=TL, s=S),
        grid=(B, L // TL),
        in_specs=[
            pl.BlockSpec((1, 1, META_LANES), lambda b, j: (b, 0, 0)),
            pl.BlockSpec((1, TL, D), lambda b, j: (b, j, 0)),
            pl.BlockSpec((1, S, D), lambda b, j: (b, 0, 0)),
            pl.BlockSpec((D, D), lambda b, j: (0, 0)),
            pl.BlockSpec((1, D), lambda b, j: (0, 0)),
        ],
        out_specs=pl.BlockSpec((1, TL, D), lambda b, j: (b, j, 0)),
        out_shape=jax.ShapeDtypeStruct((B, L, D), jnp.float32),
    )(meta, words_emb, sproj, wa, bias)
    return out
